# baseline (speedup 1.0000x reference)
import functools

import jax
import jax.numpy as jnp
from jax import lax
from jax.experimental import pallas as pl
from jax.experimental.pallas import tpu as pltpu
from jax.experimental.pallas import tpu_sc as plsc

B = 8
N = 2048
K = 256
LANES = 16


def _sim_argmax_body(d0_ref, d1_ref, sim_hbm, m0_ref, m1_ref,
                     simbuf, sems):
    b = pl.program_id(0)
    slot = lax.rem(b, 2)

    H = N // 2

    def _half_copy(s, bb, h):
        return pltpu.make_async_copy(
            simbuf.at[s, pl.ds(h * H, H)],
            sim_hbm.at[bb, pl.ds(h * H, H)],
            sems.at[s, h])

    @pl.when(b >= 2)
    def _():
        _half_copy(slot, b - 2, 0).wait()
        _half_copy(slot, b - 2, 1).wait()

    d1 = d1_ref[0]
    n1 = jnp.sqrt(jnp.sum(d1 * d1, axis=1, keepdims=True))
    d1n = d1 / jnp.maximum(n1, 1e-12)

    d0 = d0_ref[0]
    n0 = jnp.sqrt(jnp.sum(d0 * d0, axis=1, keepdims=True))
    d0n = d0 / jnp.maximum(n0, 1e-12)

    sim = lax.dot_general(d0n, d1n, (((1,), (1,)), ((), ())),
                          preferred_element_type=jnp.float32)
    simbuf[slot] = sim
    _half_copy(slot, b, 0).start()
    _half_copy(slot, b, 1).start()

    m0_ref[0] = jnp.argmax(sim, axis=1).astype(jnp.int32).reshape(1, N)
    m1_ref[0] = jnp.argmax(sim, axis=0).astype(jnp.int32).reshape(1, N)

    @pl.when(b == B - 1)
    def _():
        _half_copy(1 - slot, b - 1, 0).wait()
        _half_copy(1 - slot, b - 1, 1).wait()
        _half_copy(slot, b, 0).wait()
        _half_copy(slot, b, 1).wait()


def _sim_and_raw_matches(d0, d1):
    out_shape = (
        jax.ShapeDtypeStruct((B, N, N), jnp.float32),
        jax.ShapeDtypeStruct((B, 1, N), jnp.int32),
        jax.ShapeDtypeStruct((B, 1, N), jnp.int32),
    )
    return pl.pallas_call(
        _sim_argmax_body,
        grid=(B,),
        in_specs=[
            pl.BlockSpec((1, N, K), lambda b: (b, 0, 0)),
            pl.BlockSpec((1, N, K), lambda b: (b, 0, 0)),
        ],
        out_specs=[
            pl.BlockSpec(memory_space=pl.ANY),
            pl.BlockSpec((1, 1, N), lambda b: (b, 0, 0)),
            pl.BlockSpec((1, 1, N), lambda b: (b, 0, 0)),
        ],
        out_shape=out_shape,
        scratch_shapes=[
            pltpu.VMEM((2, N, N), jnp.float32),
            pltpu.SemaphoreType.DMA((2, 2)),
        ],
        compiler_params=pltpu.CompilerParams(
            dimension_semantics=("arbitrary",)),
    )(d0, d1)


def _mutual_check_sc(flat):
    info = plsc.get_sparse_core_info()
    nc, ns = info.num_cores, info.num_subcores
    nw = nc * ns
    total = 2 * B * N
    per = total // nw
    half_sz = B * N
    mesh = plsc.VectorSubcoreMesh(core_axis_name="c", subcore_axis_name="s")

    @functools.partial(
        pl.kernel,
        mesh=mesh,
        compiler_params=pltpu.CompilerParams(needs_layout_passes=False),
        out_type=(
            jax.ShapeDtypeStruct((total,), jnp.int32),
            jax.ShapeDtypeStruct((total,), jnp.float32),
        ),
        scratch_types=[
            pltpu.VMEM((N,), jnp.int32),
            pltpu.VMEM((per,), jnp.int32),
            pltpu.VMEM((per,), jnp.int32),
            pltpu.VMEM((per,), jnp.float32),
        ],
    )
    def k(m_hbm, outm_hbm, outs_hbm, table_v, idx_v, om_v, os_v):
        wid = lax.axis_index("s") * nc + lax.axis_index("c")
        self_off = wid * per
        side = self_off // half_sz
        pos_in_side = self_off - side * half_sz
        batch = pos_in_side // N
        base = pos_in_side - batch * N
        tbl_off = (1 - side) * half_sz + batch * N

        pltpu.sync_copy(m_hbm.at[pl.ds(tbl_off, N)], table_v)
        pltpu.sync_copy(m_hbm.at[pl.ds(self_off, per)], idx_v)

        def body(v, carry):
            idx = idx_v[pl.ds(v * LANES, LANES)]
            loop = plsc.load_gather(table_v, [idx])
            pos = lax.iota(jnp.int32, LANES) + (base + v * LANES)
            keep = loop == pos
            om_v[pl.ds(v * LANES, LANES)] = jnp.where(keep, idx, -1)
            os_v[pl.ds(v * LANES, LANES)] = jnp.where(
                keep, jnp.float32(1.0), jnp.float32(0.0))
            return carry

        lax.fori_loop(0, per // LANES, body, 0)

        pltpu.sync_copy(om_v, outm_hbm.at[pl.ds(self_off, per)])
        pltpu.sync_copy(os_v, outs_hbm.at[pl.ds(self_off, per)])

    return k(flat)


def kernel(descriptors0, descriptors1):
    sim, m0r, m1r = _sim_and_raw_matches(descriptors0, descriptors1)
    flat = jnp.concatenate([m0r.reshape(-1), m1r.reshape(-1)])
    mflat, sflat = _mutual_check_sc(flat)
    half = B * N
    matches0 = mflat[:half].reshape(B, N)
    matches1 = mflat[half:].reshape(B, N)
    mscores0 = sflat[:half].reshape(B, N)
    mscores1 = sflat[half:].reshape(B, N)
    return (matches0, matches1, mscores0, mscores1, sim)


# speedup vs baseline: 1.0123x; 1.0123x over previous
import functools

import jax
import jax.numpy as jnp
from jax import lax
from jax.experimental import pallas as pl
from jax.experimental.pallas import tpu as pltpu
from jax.experimental.pallas import tpu_sc as plsc

B = 8
N = 2048
K = 256
LANES = 16


def _sim_argmax_body(d0_ref, d1_ref, sim_ref, m0_ref, m1_ref):
    d1 = d1_ref[0]
    n1 = jnp.sqrt(jnp.sum(d1 * d1, axis=1, keepdims=True))
    d1n = d1 / jnp.maximum(n1, 1e-12)

    d0 = d0_ref[0]
    n0 = jnp.sqrt(jnp.sum(d0 * d0, axis=1, keepdims=True))
    d0n = d0 / jnp.maximum(n0, 1e-12)

    sim = lax.dot_general(d0n, d1n, (((1,), (1,)), ((), ())),
                          preferred_element_type=jnp.float32)
    sim_ref[0] = sim

    m0_ref[0] = jnp.argmax(sim, axis=1).astype(jnp.int32).reshape(1, N)
    m1_ref[0] = jnp.argmax(sim, axis=0).astype(jnp.int32).reshape(1, N)


def _sim_and_raw_matches(d0, d1):
    out_shape = (
        jax.ShapeDtypeStruct((B, N, N), jnp.float32),
        jax.ShapeDtypeStruct((B, 1, N), jnp.int32),
        jax.ShapeDtypeStruct((B, 1, N), jnp.int32),
    )
    return pl.pallas_call(
        _sim_argmax_body,
        grid=(B,),
        in_specs=[
            pl.BlockSpec((1, N, K), lambda b: (b, 0, 0)),
            pl.BlockSpec((1, N, K), lambda b: (b, 0, 0)),
        ],
        out_specs=[
            pl.BlockSpec((1, N, N), lambda b: (b, 0, 0)),
            pl.BlockSpec((1, 1, N), lambda b: (b, 0, 0)),
            pl.BlockSpec((1, 1, N), lambda b: (b, 0, 0)),
        ],
        out_shape=out_shape,
        compiler_params=pltpu.CompilerParams(
            dimension_semantics=("arbitrary",)),
    )(d0, d1)


def _mutual_check_sc(flat):
    info = plsc.get_sparse_core_info()
    nc, ns = info.num_cores, info.num_subcores
    nw = nc * ns
    total = 2 * B * N
    per = total // nw
    half_sz = B * N
    mesh = plsc.VectorSubcoreMesh(core_axis_name="c", subcore_axis_name="s")

    @functools.partial(
        pl.kernel,
        mesh=mesh,
        compiler_params=pltpu.CompilerParams(needs_layout_passes=False),
        out_type=(
            jax.ShapeDtypeStruct((total,), jnp.int32),
            jax.ShapeDtypeStruct((total,), jnp.float32),
        ),
        scratch_types=[
            pltpu.VMEM((N,), jnp.int32),
            pltpu.VMEM((per,), jnp.int32),
            pltpu.VMEM((per,), jnp.int32),
            pltpu.VMEM((per,), jnp.float32),
        ],
    )
    def k(m_hbm, outm_hbm, outs_hbm, table_v, idx_v, om_v, os_v):
        wid = lax.axis_index("s") * nc + lax.axis_index("c")
        self_off = wid * per
        side = self_off // half_sz
        pos_in_side = self_off - side * half_sz
        batch = pos_in_side // N
        base = pos_in_side - batch * N
        tbl_off = (1 - side) * half_sz + batch * N

        pltpu.sync_copy(m_hbm.at[pl.ds(tbl_off, N)], table_v)
        pltpu.sync_copy(m_hbm.at[pl.ds(self_off, per)], idx_v)

        def body(v, carry):
            idx = idx_v[pl.ds(v * LANES, LANES)]
            loop = plsc.load_gather(table_v, [idx])
            pos = lax.iota(jnp.int32, LANES) + (base + v * LANES)
            keep = loop == pos
            om_v[pl.ds(v * LANES, LANES)] = jnp.where(keep, idx, -1)
            os_v[pl.ds(v * LANES, LANES)] = jnp.where(
                keep, jnp.float32(1.0), jnp.float32(0.0))
            return carry

        lax.fori_loop(0, per // LANES, body, 0)

        pltpu.sync_copy(om_v, outm_hbm.at[pl.ds(self_off, per)])
        pltpu.sync_copy(os_v, outs_hbm.at[pl.ds(self_off, per)])

    return k(flat)


def kernel(descriptors0, descriptors1):
    sim, m0r, m1r = _sim_and_raw_matches(descriptors0, descriptors1)
    flat = jnp.concatenate([m0r.reshape(-1), m1r.reshape(-1)])
    mflat, sflat = _mutual_check_sc(flat)
    half = B * N
    matches0 = mflat[:half].reshape(B, N)
    matches1 = mflat[half:].reshape(B, N)
    mscores0 = sflat[:half].reshape(B, N)
    mscores1 = sflat[half:].reshape(B, N)
    return (matches0, matches1, mscores0, mscores1, sim)
